# Initial kernel scaffold; baseline (speedup 1.0000x reference)
import jax
import jax.numpy as jnp
from jax.experimental import pallas as pl


def kernel(positions, atomic_numbers, edge_index, batch, W_send, W_recv, rt_weights, W1, b1, W2, b2, W3, b3):
    raise NotImplementedError("write your pallas kernel here")


# speedup vs baseline: 1.0007x; 1.0007x over previous
import jax
import jax.numpy as jnp
import numpy as np
from jax.experimental import pallas as pl

NZ = 4
D_NE = 2
D_EE = D_NE * D_NE
N_R = 8
N_RE = 8
L_MAX = 2
R_CUT = 5.5
MP_NORM = 1.0 / (10.0 ** 0.5)
N_NODES = 10000
N_EDGES = 160000
N_ANG = [1, 3, 6]
N_INPUT = N_RE * D_EE * (L_MAX + 1)
HID = 32

_BLK = 2000


def _mlp_body(bf_ref, w1_ref, b1_ref, w2_ref, b2_ref, w3_ref, b3_ref, out_ref):
    i = pl.program_id(0)
    bf = bf_ref[...]
    h = bf @ w1_ref[...] + b1_ref[...][None, :]
    h = h * (1.0 / (1.0 + jnp.exp(-h)))
    h = h @ w2_ref[...] + b2_ref[...][None, :]
    h = h * (1.0 / (1.0 + jnp.exp(-h)))
    e = h @ w3_ref[...] + b3_ref[...][None, :]

    @pl.when(i == 0)
    def _():
        out_ref[...] = jnp.zeros_like(out_ref)

    out_ref[...] += jnp.sum(e).reshape(1, 1)


def kernel(positions, atomic_numbers, edge_index, batch, W_send, W_recv, rt_weights, W1, b1, W2, b2, W3, b3):
    node_enc = jax.nn.one_hot(atomic_numbers, NZ, dtype=positions.dtype)
    emb_s = node_enc @ W_send
    emb_r = node_enc @ W_recv
    snd = edge_index[0]
    rcv = edge_index[1]
    edge_enc = (emb_s[snd][:, :, None] * emb_r[rcv][:, None, :]).reshape(-1, D_EE)
    vec = positions[rcv] - positions[snd]
    d2 = jnp.sum(vec * vec, axis=-1)
    d = jnp.sqrt(d2 + 1e-12)
    u = vec / d[:, None]
    n = jnp.arange(1, N_R + 1, dtype=positions.dtype)
    rbf = jnp.sqrt(2.0 / R_CUT) * jnp.sin(n[None, :] * jnp.pi * d[:, None] / R_CUT) / d[:, None]
    fcut = 0.5 * (jnp.cos(jnp.pi * jnp.clip(d, 0.0, R_CUT) / R_CUT) + 1.0) * (d < R_CUT).astype(positions.dtype)
    rbf = rbf * fcut[:, None]
    x, y, z = u[:, 0], u[:, 1], u[:, 2]
    angs = [
        jnp.ones((u.shape[0], 1), dtype=u.dtype),
        jnp.stack([x, y, z], axis=-1),
        jnp.stack([x * x, x * y, x * z, y * y, y * z, z * z], axis=-1),
    ]
    blocks = []
    for l in range(L_MAX + 1):
        rt_l = jnp.einsum('en,nkd->ekd', rbf, rt_weights[l])
        edge_feat = rt_l * edge_enc[:, None, :]
        feat = edge_feat[:, :, :, None] * angs[l][:, None, None, :]
        A_l = jax.ops.segment_sum(feat.reshape(feat.shape[0], -1), rcv, num_segments=N_NODES)
        A_l = (A_l * MP_NORM).reshape(N_NODES, N_RE, D_EE, N_ANG[l])
        B_l = jnp.sum(A_l * A_l, axis=-1)
        blocks.append(B_l.reshape(N_NODES, -1))
    Bfeat = jnp.concatenate(blocks, axis=-1)

    grid = N_NODES // _BLK
    energy = pl.pallas_call(
        _mlp_body,
        grid=(grid,),
        in_specs=[
            pl.BlockSpec((_BLK, N_INPUT), lambda i: (i, 0)),
            pl.BlockSpec((N_INPUT, HID), lambda i: (0, 0)),
            pl.BlockSpec((HID,), lambda i: (0,)),
            pl.BlockSpec((HID, HID), lambda i: (0, 0)),
            pl.BlockSpec((HID,), lambda i: (0,)),
            pl.BlockSpec((HID, 1), lambda i: (0, 0)),
            pl.BlockSpec((1,), lambda i: (0,)),
        ],
        out_specs=pl.BlockSpec((1, 1), lambda i: (0, 0)),
        out_shape=jax.ShapeDtypeStruct((1, 1), jnp.float32),
    )(Bfeat, W1, b1, W2, b2, W3, b3)
    return energy[:, 0]


# speedup vs baseline: 3.1954x; 3.1931x over previous
import functools

import jax
import jax.numpy as jnp
import numpy as np
from jax import lax
from jax.experimental import pallas as pl
from jax.experimental.pallas import tpu as pltpu
from jax.experimental.pallas import tpu_sc as plsc

NZ = 4
D_NE = 2
D_EE = D_NE * D_NE
N_R = 8
N_RE = 8
L_MAX = 2
R_CUT = 5.5
MP_NORM = 1.0 / (10.0 ** 0.5)
N_NODES = 10000
N_EDGES = 160000
N_ANG = [1, 3, 6]
N_INPUT = N_RE * D_EE * (L_MAX + 1)
HID = 32

NPC = N_R // 2
F = NPC * D_EE * 10
NSUB = 16
EPT = N_EDGES // NSUB
CH = 80
NCHUNK = EPT // CH
NGRP = CH // 16
ROWS_PER_SUB = 632
N_ACC = ROWS_PER_SUB * NSUB

_PI = float(np.pi)
_HPI = float(np.pi / 2.0)
_SQ2RC = float(np.sqrt(2.0 / R_CUT))

_NBLK = 1264


def _C16(v):
    return jnp.full((16,), v, jnp.int32)


def _round_bf16(v):
    u = plsc.bitcast(v, jnp.int32)
    u = (u + 0x7FFF + (lax.shift_right_logical(u, 16) & 1)) & ~0xFFFF
    return plsc.bitcast(u, jnp.float32)


def _cos_poly(t2):
    c = 1.0 / 479001600.0
    c = -1.0 / 3628800.0 + t2 * c
    c = 1.0 / 40320.0 + t2 * c
    c = -1.0 / 720.0 + t2 * c
    c = 1.0 / 24.0 + t2 * c
    c = -0.5 + t2 * c
    return 1.0 + t2 * c


def _sin_poly(t, t2):
    c = -1.0 / 39916800.0
    c = 1.0 / 362880.0 + t2 * c
    c = -1.0 / 5040.0 + t2 * c
    c = 1.0 / 120.0 + t2 * c
    c = -1.0 / 6.0 + t2 * c
    return t * (1.0 + t2 * c)


def _sc_body(ptab_hbm, ws_hbm, wr_hbm, snd_hbm, rcv_hbm, zeros_hbm, out_hbm,
             ws_v, wr_v, snd_v, rcv_v, srow_v, rrow_v, msg_v, acc_sh):
    c = lax.axis_index("c")
    s = lax.axis_index("s")

    pltpu.sync_copy(ws_hbm, ws_v)
    pltpu.sync_copy(wr_hbm, wr_v)
    pltpu.sync_copy(zeros_hbm.at[pl.ds(s * ROWS_PER_SUB, ROWS_PER_SUB)],
                    acc_sh.at[pl.ds(s * ROWS_PER_SUB, ROWS_PER_SUB)])
    plsc.subcore_barrier()

    iota = lax.iota(jnp.int32, 16)
    cm = c == 0

    def chunk_body(k, carry):
        base = pl.multiple_of(s * EPT + k * CH, 8)
        pltpu.sync_copy(snd_hbm.at[pl.ds(base, CH)], snd_v)
        pltpu.sync_copy(rcv_hbm.at[pl.ds(base, CH)], rcv_v)
        pltpu.sync_copy(ptab_hbm.at[snd_v], srow_v)
        pltpu.sync_copy(ptab_hbm.at[rcv_v], rrow_v)
        for g in range(NGRP):
            row16 = g * 16 + iota
            sx = plsc.load_gather(srow_v, [row16, _C16(0)])
            sy = plsc.load_gather(srow_v, [row16, _C16(1)])
            sz = plsc.load_gather(srow_v, [row16, _C16(2)])
            zs = plsc.load_gather(srow_v, [row16, _C16(3)])
            rx = plsc.load_gather(rrow_v, [row16, _C16(0)])
            ry = plsc.load_gather(rrow_v, [row16, _C16(1)])
            rz = plsc.load_gather(rrow_v, [row16, _C16(2)])
            zr = plsc.load_gather(rrow_v, [row16, _C16(3)])
            zsi = lax.convert_element_type(zs, jnp.int32) * 2
            zri = lax.convert_element_type(zr, jnp.int32) * 2
            es0 = plsc.load_gather(ws_v, [zsi])
            es1 = plsc.load_gather(ws_v, [zsi + 1])
            er0 = plsc.load_gather(wr_v, [zri])
            er1 = plsc.load_gather(wr_v, [zri + 1])

            vx = rx - sx
            vy = ry - sy
            vz = rz - sz
            d2 = vx * vx + vy * vy + vz * vz + 1e-12
            y = plsc.bitcast(0x5F3759DF - lax.shift_right_logical(
                plsc.bitcast(d2, jnp.int32), 1), jnp.float32)
            y = y * (1.5 - 0.5 * d2 * y * y)
            y = y * (1.5 - 0.5 * d2 * y * y)
            y = y * (1.5 - 0.5 * d2 * y * y)
            rs = y
            d = d2 * rs
            ux = vx * rs
            uy = vy * rs
            uz = vz * rs

            x = jnp.minimum(d, R_CUT) * (_PI / R_CUT)
            wv = jnp.minimum(x, _PI - x)
            t2 = wv * wv
            s1 = _sin_poly(wv, t2)
            cp = _cos_poly(t2)
            c1 = jnp.where(x <= _HPI, cp, -cp)
            fcut = jnp.where(d < R_CUT, 0.5 * (c1 + 1.0), 0.0)
            inv_d = rs * (rs * d)
            wq = (_SQ2RC * inv_d) * (fcut * s1)

            c2 = c1 + c1
            q1 = c2
            q2 = c2 * q1 - 1.0
            q3 = c2 * q2 - q1
            q4 = c2 * q3 - q2
            q5 = c2 * q4 - q3
            q6 = c2 * q5 - q4
            q7 = c2 * q6 - q5
            r0 = jnp.where(cm, wq, wq * q4)
            r1 = jnp.where(cm, wq * q1, wq * q5)
            r2 = jnp.where(cm, wq * q2, wq * q6)
            r3 = jnp.where(cm, wq * q3, wq * q7)
            r = [_round_bf16(v) for v in (r0, r1, r2, r3)]

            enc = (es0 * er0, es0 * er1, es1 * er0, es1 * er1)
            ang = (None, ux, uy, uz, ux * ux, ux * uy, ux * uz,
                   uy * uy, uy * uz, uz * uz)

            row = row16
            for nl in range(NPC):
                for dd in range(D_EE):
                    tnd = r[nl] * enc[dd]
                    fbase = (nl * D_EE + dd) * 10
                    for a in range(10):
                        val = tnd if a == 0 else tnd * ang[a]
                        plsc.store_scatter(msg_v, [row, _C16(fbase + a)], val)
        pltpu.sync_copy(msg_v, acc_sh.at[rcv_v], add=True)
        return carry

    lax.fori_loop(0, NCHUNK, chunk_body, 0)
    plsc.subcore_barrier()
    pltpu.sync_copy(acc_sh.at[pl.ds(s * ROWS_PER_SUB, ROWS_PER_SUB)],
                    out_hbm.at[c, pl.ds(s * ROWS_PER_SUB, ROWS_PER_SUB)])


def _sc_scatter(ptab, ws, wr, snd, rcv, zeros):
    mesh = plsc.VectorSubcoreMesh(core_axis_name="c", subcore_axis_name="s")
    return pl.kernel(
        _sc_body,
        out_type=jax.ShapeDtypeStruct((2, N_ACC, F), jnp.float32),
        mesh=mesh,
        compiler_params=pltpu.CompilerParams(
            needs_layout_passes=False, use_tc_tiling_on_sc=False),
        scratch_types=[
            pltpu.VMEM((NZ * D_NE,), jnp.float32),
            pltpu.VMEM((NZ * D_NE,), jnp.float32),
            pltpu.VMEM((CH,), jnp.int32),
            pltpu.VMEM((CH,), jnp.int32),
            pltpu.VMEM((CH, 8), jnp.float32),
            pltpu.VMEM((CH, 8), jnp.float32),
            pltpu.VMEM((CH, F), jnp.float32),
            pltpu.VMEM_SHARED((N_ACC, F), jnp.float32),
        ],
    )(ptab, ws, wr, snd, rcv, zeros)


def _tc_body(s2_ref, r_ref, p_ref, w1_ref, b1_ref, w2_ref, b2_ref,
             w3_ref, b3_ref, out_ref):
    i = pl.program_id(0)
    hp = lax.Precision.HIGHEST

    def dot(a, b):
        return jnp.dot(a, b, precision=hp, preferred_element_type=jnp.float32)

    def bf(v):
        return v.astype(jnp.bfloat16).astype(jnp.float32)

    sfull = jnp.concatenate([s2_ref[0], s2_ref[1]], axis=-1)
    A = dot(sfull, r_ref[...])
    B = dot(A * A, p_ref[...])
    h = dot(bf(B), bf(w1_ref[...])) + b1_ref[...][None, :]
    h = h * (1.0 / (1.0 + jnp.exp(-h)))
    h = dot(bf(h), bf(w2_ref[...])) + b2_ref[...][None, :]
    h = h * (1.0 / (1.0 + jnp.exp(-h)))
    e = dot(bf(h), bf(w3_ref[...])) + b3_ref[...][None, :]
    rowid = lax.broadcasted_iota(jnp.int32, e.shape, 0) + i * _NBLK
    e = jnp.where(rowid < N_NODES, e, 0.0)

    @pl.when(i == 0)
    def _():
        out_ref[...] = jnp.zeros_like(out_ref)

    out_ref[...] += jnp.sum(e).reshape(1, 1)


def _node_readout(S2, R, P, W1, b1, W2, b2, W3, b3):
    grid = N_ACC // _NBLK
    out = pl.pallas_call(
        _tc_body,
        grid=(grid,),
        in_specs=[
            pl.BlockSpec((2, _NBLK, F), lambda i: (0, i, 0)),
            pl.BlockSpec((2 * F, 2 * F), lambda i: (0, 0)),
            pl.BlockSpec((2 * F, N_INPUT), lambda i: (0, 0)),
            pl.BlockSpec((N_INPUT, HID), lambda i: (0, 0)),
            pl.BlockSpec((HID,), lambda i: (0,)),
            pl.BlockSpec((HID, HID), lambda i: (0, 0)),
            pl.BlockSpec((HID,), lambda i: (0,)),
            pl.BlockSpec((HID, 1), lambda i: (0, 0)),
            pl.BlockSpec((1,), lambda i: (0,)),
        ],
        out_specs=pl.BlockSpec((1, 1), lambda i: (0, 0)),
        out_shape=jax.ShapeDtypeStruct((1, 1), jnp.float32),
    )(S2, R, P, W1, b1, W2, b2, W3, b3)
    return out[:, 0]


def _build_RP(rt_weights, dtype):
    R = jnp.zeros((2 * F, 2 * F), dtype=dtype)
    P = np.zeros((2 * F, N_INPUT), dtype=np.float32)
    specs = [(0, 0, 1, 0), (1, 32, 3, 1), (2, 128, 6, 4)]
    for l, off, na, aoff in specs:
        n_i, k_i, d_i, a_i = np.meshgrid(
            np.arange(N_R), np.arange(N_RE), np.arange(D_EE), np.arange(na),
            indexing="ij")
        rows = (n_i * D_EE + d_i) * 10 + aoff + a_i
        cols = off + (k_i * D_EE + d_i) * na + a_i
        vals = MP_NORM * rt_weights[l][n_i, k_i, d_i]
        R = R.at[rows.ravel(), cols.ravel()].set(vals.ravel())
        pk, pd, pa = np.meshgrid(np.arange(N_RE), np.arange(D_EE),
                                 np.arange(na), indexing="ij")
        bcols = l * 32 + pk * D_EE + pd
        P[(off + (pk * D_EE + pd) * na + pa).ravel(), bcols.ravel()] = 1.0
    return R, jnp.asarray(P, dtype=dtype)


def kernel(positions, atomic_numbers, edge_index, batch, W_send, W_recv,
           rt_weights, W1, b1, W2, b2, W3, b3):
    f32 = positions.dtype
    ptab = jnp.zeros((N_ACC, 8), jnp.float32)
    ptab = ptab.at[:N_NODES, :3].set(positions.astype(jnp.float32))
    ptab = ptab.at[:N_NODES, 3].set(atomic_numbers.astype(jnp.float32))
    snd = edge_index[0].astype(jnp.int32)
    rcv = edge_index[1].astype(jnp.int32)
    zeros = jnp.zeros((N_ACC, F), jnp.float32)
    ws_b = W_send.reshape(-1).astype(jnp.bfloat16).astype(jnp.float32)
    wr_b = W_recv.reshape(-1).astype(jnp.bfloat16).astype(jnp.float32)
    S2 = _sc_scatter(ptab, ws_b, wr_b, snd, rcv, zeros)
    rt_b = rt_weights.astype(jnp.bfloat16).astype(jnp.float32)
    R, P = _build_RP(rt_b, jnp.float32)
    energy = _node_readout(S2, R, P, W1, b1, W2, b2, W3, b3)
    return energy.astype(f32)


# speedup vs baseline: 4.0949x; 1.2815x over previous
import functools

import jax
import jax.numpy as jnp
import numpy as np
from jax import lax
from jax.experimental import pallas as pl
from jax.experimental.pallas import tpu as pltpu
from jax.experimental.pallas import tpu_sc as plsc

NZ = 4
D_NE = 2
D_EE = D_NE * D_NE
N_R = 8
N_RE = 8
L_MAX = 2
R_CUT = 5.5
MP_NORM = 1.0 / (10.0 ** 0.5)
N_NODES = 10000
N_EDGES = 160000
N_ANG = [1, 3, 6]
N_INPUT = N_RE * D_EE * (L_MAX + 1)
HID = 32

NPC = N_R // 2
F = NPC * D_EE * 10
NSUB = 16
EPT = N_EDGES // NSUB
CH = 80
NCHUNK = EPT // CH
NGRP = CH // 16
IB = 400
CPB = IB // CH
NB = EPT // IB
ROWS_PER_SUB = 632
N_ACC = ROWS_PER_SUB * NSUB

_PI = float(np.pi)
_HPI = float(np.pi / 2.0)
_SQ2RC = float(np.sqrt(2.0 / R_CUT))

_NBLK = 1264


def _C16(v):
    return jnp.full((16,), v, jnp.int32)


def _round_bf16(v):
    u = plsc.bitcast(v, jnp.int32)
    u = (u + 0x7FFF + (lax.shift_right_logical(u, 16) & 1)) & ~0xFFFF
    return plsc.bitcast(u, jnp.float32)


def _cos_poly(t2):
    c = 1.0 / 479001600.0
    c = -1.0 / 3628800.0 + t2 * c
    c = 1.0 / 40320.0 + t2 * c
    c = -1.0 / 720.0 + t2 * c
    c = 1.0 / 24.0 + t2 * c
    c = -0.5 + t2 * c
    return 1.0 + t2 * c


def _sin_poly(t, t2):
    c = -1.0 / 39916800.0
    c = 1.0 / 362880.0 + t2 * c
    c = -1.0 / 5040.0 + t2 * c
    c = 1.0 / 120.0 + t2 * c
    c = -1.0 / 6.0 + t2 * c
    return t * (1.0 + t2 * c)


def _sc_body(ptab_hbm, ws_hbm, wr_hbm, snd_hbm, rcv_hbm, zeros_hbm, out_hbm,
             ws_v, wr_v, snd_v, rcv_v, srow_v, rrow_v, msg_v, rseg_v,
             gsem0, gsem1, ssem0, ssem1, acc_sh):
    c = lax.axis_index("c")
    s = lax.axis_index("s")
    gsem = (gsem0, gsem1)
    ssem = (ssem0, ssem1)

    pltpu.sync_copy(ws_hbm, ws_v)
    pltpu.sync_copy(wr_hbm, wr_v)
    pltpu.sync_copy(zeros_hbm.at[pl.ds(s * ROWS_PER_SUB, ROWS_PER_SUB)],
                    acc_sh.at[pl.ds(s * ROWS_PER_SUB, ROWS_PER_SUB)])
    plsc.subcore_barrier()

    iota = lax.iota(jnp.int32, 16)
    cm = c == 0

    def emit_chunk(bj):
        def gbody(g, carry):
            row16 = g * 16 + iota
            _compute_group(row16, srow_v.at[bj], rrow_v.at[bj], ws_v, wr_v,
                           msg_v.at[bj], cm)
            return carry

        lax.fori_loop(0, NGRP, gbody, 0)

    def block_body(bi, carry):
        base = pl.multiple_of(s * EPT + bi * IB, 8)
        pltpu.sync_copy(snd_hbm.at[pl.ds(base, IB)], snd_v)
        pltpu.sync_copy(rcv_hbm.at[pl.ds(base, IB)], rcv_v)
        gd = [None] * CPB
        sd = [None] * CPB

        def issue_gathers(j):
            bj = j & 1
            sidx = snd_v.at[pl.ds(j * CH, CH)]
            ridx = rcv_v.at[pl.ds(j * CH, CH)]
            g1 = pltpu.async_copy(ptab_hbm.at[sidx], srow_v.at[bj], gsem[bj])
            g2 = pltpu.async_copy(ptab_hbm.at[ridx], rrow_v.at[bj], gsem[bj])
            return (g1, g2)

        gd[0] = issue_gathers(0)
        for j in range(CPB):
            bj = j & 1
            if j + 1 < CPB:
                gd[j + 1] = issue_gathers(j + 1)
            gd[j][0].wait()
            gd[j][1].wait()
            if j >= 2:
                sd[j - 2].wait()
            emit_chunk(bj)
            for g in range(NGRP):
                rseg_v[bj, pl.ds(g * 16, 16)] = rcv_v[pl.ds(j * CH + g * 16, 16)]
            sd[j] = pltpu.async_copy(msg_v.at[bj], acc_sh.at[rseg_v.at[bj]],
                                     ssem[bj], add=True)
        sd[CPB - 2].wait()
        sd[CPB - 1].wait()
        return carry

    lax.fori_loop(0, NB, block_body, 0)
    plsc.subcore_barrier()
    pltpu.sync_copy(acc_sh.at[pl.ds(s * ROWS_PER_SUB, ROWS_PER_SUB)],
                    out_hbm.at[c, pl.ds(s * ROWS_PER_SUB, ROWS_PER_SUB)])


def _compute_group(row16, srow_v, rrow_v, ws_v, wr_v, msg_v, cm):
    if True:
        if True:
            sx = plsc.load_gather(srow_v, [row16, _C16(0)])
            sy = plsc.load_gather(srow_v, [row16, _C16(1)])
            sz = plsc.load_gather(srow_v, [row16, _C16(2)])
            zs = plsc.load_gather(srow_v, [row16, _C16(3)])
            rx = plsc.load_gather(rrow_v, [row16, _C16(0)])
            ry = plsc.load_gather(rrow_v, [row16, _C16(1)])
            rz = plsc.load_gather(rrow_v, [row16, _C16(2)])
            zr = plsc.load_gather(rrow_v, [row16, _C16(3)])
            zsi = lax.convert_element_type(zs, jnp.int32) * 2
            zri = lax.convert_element_type(zr, jnp.int32) * 2
            es0 = plsc.load_gather(ws_v, [zsi])
            es1 = plsc.load_gather(ws_v, [zsi + 1])
            er0 = plsc.load_gather(wr_v, [zri])
            er1 = plsc.load_gather(wr_v, [zri + 1])

            vx = rx - sx
            vy = ry - sy
            vz = rz - sz
            d2 = vx * vx + vy * vy + vz * vz + 1e-12
            y = plsc.bitcast(0x5F3759DF - lax.shift_right_logical(
                plsc.bitcast(d2, jnp.int32), 1), jnp.float32)
            y = y * (1.5 - 0.5 * d2 * y * y)
            y = y * (1.5 - 0.5 * d2 * y * y)
            y = y * (1.5 - 0.5 * d2 * y * y)
            rs = y
            d = d2 * rs
            ux = vx * rs
            uy = vy * rs
            uz = vz * rs

            x = jnp.minimum(d, R_CUT) * (_PI / R_CUT)
            wv = jnp.minimum(x, _PI - x)
            t2 = wv * wv
            s1 = _sin_poly(wv, t2)
            cp = _cos_poly(t2)
            c1 = jnp.where(x <= _HPI, cp, -cp)
            fcut = jnp.where(d < R_CUT, 0.5 * (c1 + 1.0), 0.0)
            inv_d = rs * (rs * d)
            wq = (_SQ2RC * inv_d) * (fcut * s1)

            c2 = c1 + c1
            q1 = c2
            q2 = c2 * q1 - 1.0
            q3 = c2 * q2 - q1
            q4 = c2 * q3 - q2
            q5 = c2 * q4 - q3
            q6 = c2 * q5 - q4
            q7 = c2 * q6 - q5
            r0 = jnp.where(cm, wq, wq * q4)
            r1 = jnp.where(cm, wq * q1, wq * q5)
            r2 = jnp.where(cm, wq * q2, wq * q6)
            r3 = jnp.where(cm, wq * q3, wq * q7)
            r = [_round_bf16(v) for v in (r0, r1, r2, r3)]

            enc = (es0 * er0, es0 * er1, es1 * er0, es1 * er1)
            ang = (None, ux, uy, uz, ux * ux, ux * uy, ux * uz,
                   uy * uy, uy * uz, uz * uz)

            row = row16
            for nl in range(NPC):
                for dd in range(D_EE):
                    tnd = r[nl] * enc[dd]
                    fbase = (nl * D_EE + dd) * 10
                    for a in range(10):
                        val = tnd if a == 0 else tnd * ang[a]
                        plsc.store_scatter(msg_v, [row, _C16(fbase + a)], val)


def _sc_scatter(ptab, ws, wr, snd, rcv, zeros):
    mesh = plsc.VectorSubcoreMesh(core_axis_name="c", subcore_axis_name="s")
    return pl.kernel(
        _sc_body,
        out_type=jax.ShapeDtypeStruct((2, N_ACC, F), jnp.float32),
        mesh=mesh,
        compiler_params=pltpu.CompilerParams(
            needs_layout_passes=False, use_tc_tiling_on_sc=False),
        scratch_types=[
            pltpu.VMEM((NZ * D_NE,), jnp.float32),
            pltpu.VMEM((NZ * D_NE,), jnp.float32),
            pltpu.VMEM((IB,), jnp.int32),
            pltpu.VMEM((IB,), jnp.int32),
            pltpu.VMEM((2, CH, 8), jnp.float32),
            pltpu.VMEM((2, CH, 8), jnp.float32),
            pltpu.VMEM((2, CH, F), jnp.float32),
            pltpu.VMEM((2, CH), jnp.int32),
            pltpu.SemaphoreType.DMA,
            pltpu.SemaphoreType.DMA,
            pltpu.SemaphoreType.DMA,
            pltpu.SemaphoreType.DMA,
            pltpu.VMEM_SHARED((N_ACC, F), jnp.float32),
        ],
    )(ptab, ws, wr, snd, rcv, zeros)


def _tc_body(s2_ref, r_ref, p_ref, w1_ref, b1_ref, w2_ref, b2_ref,
             w3_ref, b3_ref, out_ref):
    i = pl.program_id(0)
    hp = lax.Precision.HIGHEST

    def dot(a, b):
        return jnp.dot(a, b, precision=hp, preferred_element_type=jnp.float32)

    def bf(v):
        return v.astype(jnp.bfloat16).astype(jnp.float32)

    sfull = jnp.concatenate([s2_ref[0], s2_ref[1]], axis=-1)
    A = dot(sfull, r_ref[...])
    B = dot(A * A, p_ref[...])
    h = dot(bf(B), bf(w1_ref[...])) + b1_ref[...][None, :]
    h = h * (1.0 / (1.0 + jnp.exp(-h)))
    h = dot(bf(h), bf(w2_ref[...])) + b2_ref[...][None, :]
    h = h * (1.0 / (1.0 + jnp.exp(-h)))
    e = dot(bf(h), bf(w3_ref[...])) + b3_ref[...][None, :]
    rowid = lax.broadcasted_iota(jnp.int32, e.shape, 0) + i * _NBLK
    e = jnp.where(rowid < N_NODES, e, 0.0)

    @pl.when(i == 0)
    def _():
        out_ref[...] = jnp.zeros_like(out_ref)

    out_ref[...] += jnp.sum(e).reshape(1, 1)


def _node_readout(S2, R, P, W1, b1, W2, b2, W3, b3):
    grid = N_ACC // _NBLK
    out = pl.pallas_call(
        _tc_body,
        grid=(grid,),
        in_specs=[
            pl.BlockSpec((2, _NBLK, F), lambda i: (0, i, 0)),
            pl.BlockSpec((2 * F, 2 * F), lambda i: (0, 0)),
            pl.BlockSpec((2 * F, N_INPUT), lambda i: (0, 0)),
            pl.BlockSpec((N_INPUT, HID), lambda i: (0, 0)),
            pl.BlockSpec((HID,), lambda i: (0,)),
            pl.BlockSpec((HID, HID), lambda i: (0, 0)),
            pl.BlockSpec((HID,), lambda i: (0,)),
            pl.BlockSpec((HID, 1), lambda i: (0, 0)),
            pl.BlockSpec((1,), lambda i: (0,)),
        ],
        out_specs=pl.BlockSpec((1, 1), lambda i: (0, 0)),
        out_shape=jax.ShapeDtypeStruct((1, 1), jnp.float32),
    )(S2, R, P, W1, b1, W2, b2, W3, b3)
    return out[:, 0]


def _build_RP(rt_weights, dtype):
    R = jnp.zeros((2 * F, 2 * F), dtype=dtype)
    P = np.zeros((2 * F, N_INPUT), dtype=np.float32)
    specs = [(0, 0, 1, 0), (1, 32, 3, 1), (2, 128, 6, 4)]
    for l, off, na, aoff in specs:
        n_i, k_i, d_i, a_i = np.meshgrid(
            np.arange(N_R), np.arange(N_RE), np.arange(D_EE), np.arange(na),
            indexing="ij")
        rows = (n_i * D_EE + d_i) * 10 + aoff + a_i
        cols = off + (k_i * D_EE + d_i) * na + a_i
        vals = MP_NORM * rt_weights[l][n_i, k_i, d_i]
        R = R.at[rows.ravel(), cols.ravel()].set(vals.ravel())
        pk, pd, pa = np.meshgrid(np.arange(N_RE), np.arange(D_EE),
                                 np.arange(na), indexing="ij")
        bcols = l * 32 + pk * D_EE + pd
        P[(off + (pk * D_EE + pd) * na + pa).ravel(), bcols.ravel()] = 1.0
    return R, jnp.asarray(P, dtype=dtype)


def kernel(positions, atomic_numbers, edge_index, batch, W_send, W_recv,
           rt_weights, W1, b1, W2, b2, W3, b3):
    f32 = positions.dtype
    ptab = jnp.zeros((N_ACC, 8), jnp.float32)
    ptab = ptab.at[:N_NODES, :3].set(positions.astype(jnp.float32))
    ptab = ptab.at[:N_NODES, 3].set(atomic_numbers.astype(jnp.float32))
    snd = edge_index[0].astype(jnp.int32)
    rcv = edge_index[1].astype(jnp.int32)
    zeros = jnp.zeros((N_ACC, F), jnp.float32)
    ws_b = W_send.reshape(-1).astype(jnp.bfloat16).astype(jnp.float32)
    wr_b = W_recv.reshape(-1).astype(jnp.bfloat16).astype(jnp.float32)
    S2 = _sc_scatter(ptab, ws_b, wr_b, snd, rcv, zeros)
    rt_b = rt_weights.astype(jnp.bfloat16).astype(jnp.float32)
    R, P = _build_RP(rt_b, jnp.float32)
    energy = _node_readout(S2, R, P, W1, b1, W2, b2, W3, b3)
    return energy.astype(f32)
